# baseline (speedup 1.0000x reference)
import functools
import jax
import jax.numpy as jnp
from jax import lax
from jax.experimental import pallas as pl
from jax.experimental.pallas import tpu as pltpu
from jax.experimental.pallas import tpu_sc as plsc

N = 50000
E = 800000
D = 64
EPS = 1e-5

NW = 32
EPW = E // NW
C2 = 448
C2R = EPW - (EPW // C2) * C2
C4 = 1600
NPAD = 50048

BE = 6400
NB3 = E // BE

BN = 2000
NB1 = N // BN
BK = 2944
NB5 = NPAD // BK

NEGINF = float("-inf")
POSINF = float("inf")


def _k1_body(xc_ref, w_ref, b_ref, ab_ref):
    h = jnp.dot(xc_ref[...], w_ref[...], preferred_element_type=jnp.float32)
    ab_ref[...] = h + b_ref[...]


def _k1(xc, wcat, bcat):
    return pl.pallas_call(
        _k1_body,
        grid=(NB1,),
        in_specs=[
            pl.BlockSpec((BN, 67), lambda i: (i, 0)),
            pl.BlockSpec((67, 2 * D), lambda i: (0, 0)),
            pl.BlockSpec((1, 2 * D), lambda i: (0, 0)),
        ],
        out_specs=pl.BlockSpec((BN, 2 * D), lambda i: (i, 0)),
        out_shape=jax.ShapeDtypeStruct((N, 2 * D), jnp.float32),
    )(xc, wcat, bcat)


def _k2(dst, src, ab):
    mesh = plsc.VectorSubcoreMesh(core_axis_name="c", subcore_axis_name="s")

    @functools.partial(
        pl.kernel,
        out_type=jax.ShapeDtypeStruct((E, 2 * D), jnp.float32),
        mesh=mesh,
        scratch_types=[
            pltpu.VMEM((C2,), jnp.int32),
            pltpu.VMEM((C2,), jnp.int32),
            pltpu.VMEM((C2, 2 * D), jnp.float32),
            pltpu.VMEM((C2, 2 * D), jnp.float32),
            pltpu.SemaphoreType.DMA,
            pltpu.SemaphoreType.DMA,
        ],
    )
    def k2(dst_hbm, src_hbm, ab_hbm, g_hbm, dstv, srcv, ra, rb, s1, s2):
        wid = lax.axis_index("s") * 2 + lax.axis_index("c")
        base = wid * EPW

        def do_chunk(off, sz):
            sl = pl.ds(0, sz)
            pltpu.sync_copy(dst_hbm.at[pl.ds(off, sz)], dstv.at[sl])
            pltpu.sync_copy(src_hbm.at[pl.ds(off, sz)], srcv.at[sl])
            cpa = pltpu.async_copy(ab_hbm.at[dstv.at[sl]], ra.at[sl], s1)
            cpb = pltpu.async_copy(ab_hbm.at[srcv.at[sl]], rb.at[sl], s2)
            cpa.wait()
            cpb.wait()

            def addrow(r, _):
                for c in range(D // 16):
                    plsc.addupdate(ra.at[r, pl.ds(c * 16, 16)],
                                   rb[r, pl.ds(D + c * 16, 16)])
                return 0
            lax.fori_loop(0, sz, addrow, 0, unroll=False)
            pltpu.sync_copy(ra.at[sl], g_hbm.at[pl.ds(off, sz)])

        def chunk(i, _):
            do_chunk(base + i * C2, C2)
            return 0

        lax.fori_loop(0, EPW // C2, chunk, 0, unroll=False)
        if C2R:
            do_chunk(base + (EPW // C2) * C2, C2R)

    return k2(dst, src, ab)


def _k3_body(g_ref, ew_ref, w2_ref, w3_ref, aff_ref, mt_ref):
    s1 = aff_ref[0:1, :]
    t1 = aff_ref[1:2, :]
    b2 = aff_ref[2:3, :]
    s2 = aff_ref[3:4, :]
    t2 = aff_ref[4:5, :]
    b3 = aff_ref[5:6, :]
    s3 = aff_ref[6:7, :]
    t3 = aff_ref[7:8, :]
    g = g_ref[:, :D]
    h = jnp.maximum(g, 0.0) * s1 + t1
    h = jnp.dot(h, w2_ref[...], preferred_element_type=jnp.float32) + b2
    h = jnp.maximum(h, 0.0) * s2 + t2
    h = jnp.dot(h, w3_ref[...], preferred_element_type=jnp.float32) + b3
    h = jnp.maximum(h, 0.0) * s3 + t3
    ht = jnp.transpose(h, (1, 0))
    mt_ref[...] = ht * ew_ref[0]


def _k3(g, ew3, w2, w3, aff):
    return pl.pallas_call(
        _k3_body,
        grid=(NB3,),
        in_specs=[
            pl.BlockSpec((BE, 2 * D), lambda i: (i, 0)),
            pl.BlockSpec((1, 1, BE), lambda i: (i, 0, 0)),
            pl.BlockSpec((D, D), lambda i: (0, 0)),
            pl.BlockSpec((D, D), lambda i: (0, 0)),
            pl.BlockSpec((8, D), lambda i: (0, 0)),
        ],
        out_specs=pl.BlockSpec((D, BE), lambda i: (0, i)),
        out_shape=jax.ShapeDtypeStruct((D, E), jnp.float32),
    )(g, ew3, w2, w3, aff)


def _k4(dst, mt1d):
    mesh = plsc.VectorSubcoreMesh(core_axis_name="c", subcore_axis_name="s")

    @functools.partial(
        pl.kernel,
        out_type=jax.ShapeDtypeStruct((D * NPAD,), jnp.float32),
        mesh=mesh,
        scratch_types=[
            pltpu.VMEM((C4,), jnp.int32),
            pltpu.VMEM((C4,), jnp.float32),
            pltpu.VMEM((C4,), jnp.float32),
            pltpu.VMEM((NPAD,), jnp.float32),
            pltpu.VMEM((NPAD,), jnp.float32),
            pltpu.VMEM((C4,), jnp.int32),
            pltpu.VMEM((C4,), jnp.float32),
            pltpu.VMEM((C4,), jnp.float32),
            pltpu.VMEM((C4,), jnp.int32),
            pltpu.VMEM((C4,), jnp.float32),
            pltpu.VMEM((C4,), jnp.float32),
        ],
        compiler_params=pltpu.CompilerParams(needs_layout_passes=False),
    )
    def k4(dst_hbm, mt_hbm, out_hbm, dstv, v0, v1, t0, t1,
           oda, o0a, o1a, odb, o0b, o1b):
        wid = lax.axis_index("s") * 2 + lax.axis_index("c")
        c0 = wid * 2

        def initrow(r, _):
            sl = pl.ds(r * 16, 16)
            t0[sl] = jnp.full((16,), NEGINF, jnp.float32)
            t1[sl] = jnp.full((16,), NEGINF, jnp.float32)
            return 0
        lax.fori_loop(0, NPAD // 16, initrow, 0, unroll=False)

        def chunk(i, _):
            off = i * C4
            pltpu.sync_copy(dst_hbm.at[pl.ds(off, C4)], dstv)
            pltpu.sync_copy(mt_hbm.at[pl.ds(c0 * E + off, C4)], v0)
            pltpu.sync_copy(mt_hbm.at[pl.ds((c0 + 1) * E + off, C4)], v1)

            def update(d16, x0, x1, obuf_d, obuf0, obuf1, nof):
                cur0 = plsc.load_gather(t0, [d16])
                plsc.store_scatter(t0, [d16], jnp.maximum(cur0, x0))
                cur1 = plsc.load_gather(t1, [d16])
                plsc.store_scatter(t1, [d16], jnp.maximum(cur1, x1))
                back0 = plsc.load_gather(t0, [d16])
                back1 = plsc.load_gather(t1, [d16])
                bad0 = back0 < x0
                bad1 = back1 < x1
                osl = pl.ds(nof * 16, 16)
                obuf_d[osl] = d16
                obuf0[osl] = jnp.where(bad0, x0, NEGINF)
                obuf1[osl] = jnp.where(bad1, x1, NEGINF)
                anybad = jnp.any(bad0 | bad1)
                return nof + jnp.where(anybad, jnp.int32(1), jnp.int32(0))

            def group(k, nof):
                sl = pl.ds(k * 16, 16)
                return update(dstv[sl], v0[sl], v1[sl], oda, o0a, o1a, nof)

            nof = lax.fori_loop(0, C4 // 16, group, jnp.int32(0),
                                unroll=False)

            bufs = ((oda, o0a, o1a), (odb, o0b, o1b))
            for p in range(16):
                srcb = bufs[p % 2]
                dstb = bufs[(p + 1) % 2]

                def ogroup(k, nof2):
                    sl = pl.ds(k * 16, 16)
                    return update(srcb[0][sl], srcb[1][sl], srcb[2][sl],
                                  dstb[0], dstb[1], dstb[2], nof2)

                nof = lax.fori_loop(0, nof, ogroup, jnp.int32(0),
                                    unroll=False)
            return 0

        lax.fori_loop(0, E // C4, chunk, 0, unroll=False)

        def fixrow(r, _):
            sl = pl.ds(r * 16, 16)
            for tref in (t0, t1):
                v = tref[sl]
                ok = (v > NEGINF) & (v < POSINF)
                tref[sl] = jnp.where(ok, v, 0.0)
            return 0
        lax.fori_loop(0, NPAD // 16, fixrow, 0, unroll=False)

        pltpu.sync_copy(t0, out_hbm.at[pl.ds(c0 * NPAD, NPAD)])
        pltpu.sync_copy(t1, out_hbm.at[pl.ds((c0 + 1) * NPAD, NPAD)])

    return k4(dst, mt1d)


def _k5_body(ot_ref, o_ref):
    o_ref[...] = jnp.transpose(ot_ref[...], (1, 0))


def _k5(ot):
    return pl.pallas_call(
        _k5_body,
        grid=(NB5,),
        in_specs=[pl.BlockSpec((D, BK), lambda i: (0, i))],
        out_specs=pl.BlockSpec((BK, D), lambda i: (i, 0)),
        out_shape=jax.ShapeDtypeStruct((NPAD, D), jnp.float32),
    )(ot)


def kernel(x, pos, edge_index, edge_weight, W1, b1, g1, be1, W2, b2, g2, be2,
           W3, b3, g3, be3):
    src = edge_index[0]
    dst = edge_index[1]

    bn = jnp.float32(1.0) / jnp.sqrt(jnp.float32(1.0 + EPS))
    s1 = g1 * bn
    s2 = g2 * bn
    s3 = g3 * bn

    wa = jnp.concatenate([W1[0:3] - W1[3:6], W1[6:]], axis=0)
    wb = jnp.concatenate([W1[3:6], jnp.zeros((D, D), jnp.float32)], axis=0)
    wcat = jnp.concatenate([wa, wb], axis=1)
    bcat = jnp.concatenate([b1, jnp.zeros((D,), jnp.float32)])[None, :]

    xc = jnp.concatenate([pos, x], axis=1)
    aff = jnp.stack([s1, be1, b2, s2, be2, b3, s3, be3], axis=0)
    ew3 = edge_weight.reshape(NB3, 1, BE)

    ab = _k1(xc, wcat, bcat)
    g = _k2(dst, src, ab)
    mt = _k3(g, ew3, W2, W3, aff)
    ot1 = _k4(dst, mt.reshape(D * E))
    return _k5(ot1.reshape(D, NPAD))[:N]
